# Initial kernel scaffold; baseline (speedup 1.0000x reference)
import jax
import jax.numpy as jnp
from jax.experimental import pallas as pl


def kernel(x, edge_index, batch, params):
    raise NotImplementedError("write your pallas kernel here")


# speedup vs baseline: 3.4740x; 3.4740x over previous
import functools

import jax
import jax.numpy as jnp
from jax import lax
from jax.experimental import pallas as pl
from jax.experimental.pallas import tpu as pltpu
from jax.experimental.pallas import tpu_sc as plsc

N = 10000
E = 320000
D = 128
G = 64
NC = 2
NS = 16
NW = NC * NS
EPW = E // NW
CH = 80
GC = 8
EPWP = 10240
NCH = EPWP // CH
NG = NCH // GC
NACC = 10240
RPT = NACC // NS
ZB = 128
RB = 2000
NRB = N // RB
BN_EPS = 1e-5


def _agg_body(h_hbm, src_hbm, dst_hbm, zeros_hbm, out_hbm,
              srcg, dstg, rows0, rows1, zbuf, acc, sem0, sem1):
    c = lax.axis_index("c")
    s = lax.axis_index("s")
    w = s * NC + c
    row0 = pl.multiple_of(s * RPT, ZB)

    pltpu.sync_copy(zeros_hbm, zbuf)
    for k in range(RPT // ZB):
        pltpu.sync_copy(zbuf, acc.at[pl.ds(row0 + k * ZB, ZB)])
    plsc.subcore_barrier()

    bufs = (rows0, rows1)
    sems = (sem0, sem1)

    def group(g, carry):
        g8 = pl.multiple_of(g * GC, GC)
        pltpu.sync_copy(src_hbm.at[w, pl.ds(g8, GC)], srcg)
        pltpu.sync_copy(dst_hbm.at[w, pl.ds(g8, GC)], dstg)
        pltpu.async_copy(h_hbm.at[srcg.at[0]], rows0, sem0)
        for k in range(GC):
            b, sm = bufs[k % 2], sems[k % 2]
            pltpu.make_async_copy(h_hbm.at[srcg.at[k]], b, sm).wait()
            if k + 1 < GC:
                pltpu.async_copy(h_hbm.at[srcg.at[k + 1]],
                                 bufs[(k + 1) % 2], sems[(k + 1) % 2])
            pltpu.sync_copy(b, acc.at[dstg.at[k]], add=True)
        return carry

    lax.fori_loop(0, NG, group, 0)
    plsc.subcore_barrier()

    for k in range(RPT // ZB):
        pltpu.sync_copy(acc.at[pl.ds(row0 + k * ZB, ZB)], zbuf)
        pltpu.sync_copy(zbuf, out_hbm.at[c, pl.ds(row0 + k * ZB, ZB)])


@functools.lru_cache(maxsize=None)
def _build_agg():
    mesh = plsc.VectorSubcoreMesh(core_axis_name="c", subcore_axis_name="s")
    return pl.kernel(
        _agg_body,
        out_type=jax.ShapeDtypeStruct((NC, NACC, D), jnp.float32),
        mesh=mesh,
        scratch_types=[
            pltpu.VMEM((GC, CH), jnp.int32),
            pltpu.VMEM((GC, CH), jnp.int32),
            pltpu.VMEM((CH, D), jnp.float32),
            pltpu.VMEM((CH, D), jnp.float32),
            pltpu.VMEM((ZB, D), jnp.float32),
            pltpu.VMEM_SHARED((NACC, D), jnp.float32),
            pltpu.SemaphoreType.DMA,
            pltpu.SemaphoreType.DMA,
        ],
    )


_BN_SCALE = 1.0 / (1.0 + BN_EPS) ** 0.5


def _mlp_math(x, a0, a1, onep, w1, b1, w2, b2, gam, bet):
    h = x * onep + a0 + a1
    h = jnp.maximum(jnp.dot(h, w1, preferred_element_type=jnp.float32) + b1, 0.0)
    h = jnp.maximum(jnp.dot(h, w2, preferred_element_type=jnp.float32) + b2, 0.0)
    return h * (gam * _BN_SCALE) + bet


def _mlp_block(x_ref, a0_ref, a1_ref, onep_ref, w1_ref, b1_ref, w2_ref,
               b2_ref, g_ref, be_ref, o_ref):
    o_ref[...] = _mlp_math(x_ref[...], a0_ref[...], a1_ref[...], onep_ref[...],
                           w1_ref[...], b1_ref[...], w2_ref[...], b2_ref[...],
                           g_ref[...], be_ref[...])


_ROW = lambda i: (i, 0)
_FIX = lambda i: (0, 0)


def _mlp_layer(x, a0, a1, onep, w1, b1, w2, b2, gam, bet):
    return pl.pallas_call(
        _mlp_block,
        grid=(NRB,),
        in_specs=[
            pl.BlockSpec((RB, D), _ROW),
            pl.BlockSpec((RB, D), _ROW),
            pl.BlockSpec((RB, D), _ROW),
            pl.BlockSpec((1, D), _FIX),
            pl.BlockSpec((D, D), _FIX),
            pl.BlockSpec((1, D), _FIX),
            pl.BlockSpec((D, D), _FIX),
            pl.BlockSpec((1, D), _FIX),
            pl.BlockSpec((1, D), _FIX),
            pl.BlockSpec((1, D), _FIX),
        ],
        out_specs=pl.BlockSpec((RB, D), _ROW),
        out_shape=jax.ShapeDtypeStruct((N, D), jnp.float32),
    )(x, a0, a1, onep, w1, b1, w2, b2, gam, bet)


def _final_block(x_ref, a0_ref, a1_ref, onep_ref, w1_ref, b1_ref, w2_ref,
                 b2_ref, g_ref, be_ref, batch_ref, l1w_ref, l1b_ref,
                 l2w_ref, l2b_ref, o_ref, pooled, counts):
    i = pl.program_id(0)

    @pl.when(i == 0)
    def _init():
        pooled[...] = jnp.zeros_like(pooled)
        counts[...] = jnp.zeros_like(counts)

    h = _mlp_math(x_ref[...], a0_ref[...], a1_ref[...], onep_ref[...],
                  w1_ref[...], b1_ref[...], w2_ref[...], b2_ref[...],
                  g_ref[...], be_ref[...])
    b = batch_ref[0, 0, :]
    onehot = (lax.broadcasted_iota(jnp.int32, (G, RB), 0)
              == b[None, :]).astype(jnp.float32)
    pooled[...] += jnp.dot(onehot, h, preferred_element_type=jnp.float32)
    counts[...] += jnp.sum(onehot, axis=1, keepdims=True)

    @pl.when(i == NRB - 1)
    def _head():
        p = pooled[...] / jnp.maximum(counts[...], 1.0)
        z = jnp.maximum(
            jnp.dot(p, l1w_ref[...], preferred_element_type=jnp.float32)
            + l1b_ref[...], 0.0)
        z = (jnp.dot(z, l2w_ref[...], preferred_element_type=jnp.float32)
             + l2b_ref[...])
        m = jnp.max(z, axis=-1, keepdims=True)
        e = z - m
        lse = jnp.log(jnp.sum(jnp.exp(e), axis=-1, keepdims=True))
        o_ref[...] = e - lse


def _final_layer(x, a0, a1, onep, w1, b1, w2, b2, gam, bet, batch3,
                 l1w, l1b, l2w, l2b):
    return pl.pallas_call(
        _final_block,
        grid=(NRB,),
        in_specs=[
            pl.BlockSpec((RB, D), _ROW),
            pl.BlockSpec((RB, D), _ROW),
            pl.BlockSpec((RB, D), _ROW),
            pl.BlockSpec((1, D), _FIX),
            pl.BlockSpec((D, D), _FIX),
            pl.BlockSpec((1, D), _FIX),
            pl.BlockSpec((D, D), _FIX),
            pl.BlockSpec((1, D), _FIX),
            pl.BlockSpec((1, D), _FIX),
            pl.BlockSpec((1, D), _FIX),
            pl.BlockSpec((1, 1, RB), lambda i: (i, 0, 0)),
            pl.BlockSpec((D, D), _FIX),
            pl.BlockSpec((1, D), _FIX),
            pl.BlockSpec((D, D), _FIX),
            pl.BlockSpec((1, D), _FIX),
        ],
        out_specs=pl.BlockSpec((G, D), _FIX),
        out_shape=jax.ShapeDtypeStruct((G, D), jnp.float32),
        scratch_shapes=[
            pltpu.VMEM((G, D), jnp.float32),
            pltpu.VMEM((G, D), jnp.float32),
        ],
    )(x, a0, a1, onep, w1, b1, w2, b2, gam, bet, batch3, l1w, l1b, l2w, l2b)


def kernel(x, edge_index, batch, params):
    src = edge_index[0].astype(jnp.int32).reshape(NW, EPW)
    dst = edge_index[1].astype(jnp.int32).reshape(NW, EPW)
    pad = EPWP - EPW
    src = jnp.concatenate(
        [src, jnp.zeros((NW, pad), jnp.int32)], axis=1).reshape(NW, NCH, CH)
    dst = jnp.concatenate(
        [dst, jnp.full((NW, pad), NACC - 1, jnp.int32)], axis=1
    ).reshape(NW, NCH, CH)
    batch3 = batch.astype(jnp.int32).reshape(NRB, 1, RB)
    zrows = jnp.zeros((ZB, D), jnp.float32)
    agg = _build_agg()

    h = x
    n_layers = len(params['convs'])
    for li, p in enumerate(params['convs']):
        aggs = agg(h, src, dst, zrows)[:, :N, :]
        onep = jnp.broadcast_to(1.0 + p['eps'], (1, D)).astype(jnp.float32)
        w1, w2 = p['W1'], p['W2']
        b1 = p['b1'].reshape(1, D)
        b2 = p['b2'].reshape(1, D)
        gam = p['gamma'].reshape(1, D)
        bet = p['beta'].reshape(1, D)
        if li < n_layers - 1:
            h = _mlp_layer(h, aggs[0], aggs[1], onep, w1, b1, w2, b2, gam, bet)
        else:
            return _final_layer(
                h, aggs[0], aggs[1], onep, w1, b1, w2, b2, gam, bet, batch3,
                params['lin1_W'], params['lin1_b'].reshape(1, D),
                params['lin2_W'], params['lin2_b'].reshape(1, D))


# speedup vs baseline: 3.9430x; 1.1350x over previous
import functools

import jax
import jax.numpy as jnp
from jax import lax
from jax.experimental import pallas as pl
from jax.experimental.pallas import tpu as pltpu
from jax.experimental.pallas import tpu_sc as plsc

N = 10000
E = 320000
D = 128
G = 64
NC = 2
NS = 16
NW = NC * NS
EPW = E // NW
CH = 80
GC = 8
EPWP = 10240
NCH = EPWP // CH
NG = NCH // GC
NACC = 10240
RPT = NACC // NS
ZB = 128
RB = 2000
NRB = N // RB
BN_EPS = 1e-5


def _agg_body(h_hbm, src_hbm, dst_hbm, zeros_hbm, out_hbm,
              is0, id0, is1, id1, b0, b1, b2, b3, acc,
              g0, g1, g2, g3, s0, s1, s2, s3, ix0, ix1, zsem):
    c = lax.axis_index("c")
    s = lax.axis_index("s")
    w = s * NC + c
    row0 = pl.multiple_of(s * RPT, ZB)
    bufs = [b0, b1, b2, b3]
    gsem = [g0, g1, g2, g3]
    ssem = [s0, s1, s2, s3]
    ib = [(is0, id0), (is1, id1)]
    isem = [ix0, ix1]

    pltpu.sync_copy(zeros_hbm, b0)
    for k in range(RPT // CH):
        off = pl.multiple_of(row0 + k * CH, 8)
        pltpu.async_copy(b0, acc.at[pl.ds(off, CH)], zsem)
    for k in range(RPT // CH):
        pltpu.make_async_copy(b0, acc.at[pl.ds(row0, CH)], zsem).wait()

    pltpu.sync_copy(src_hbm.at[w, pl.ds(0, GC)], is0)
    pltpu.sync_copy(dst_hbm.at[w, pl.ds(0, GC)], id0)
    pltpu.async_copy(src_hbm.at[w, pl.ds(GC, GC)], is1, ix1)
    pltpu.async_copy(dst_hbm.at[w, pl.ds(GC, GC)], id1, ix1)
    pltpu.async_copy(h_hbm.at[is0.at[0]], b0, g0)
    pltpu.async_copy(h_hbm.at[is0.at[1]], b1, g1)
    plsc.subcore_barrier()

    def do_chunk(gsl, k, first_group):
        slot = k % 4
        fslot = (k + 2) % 4
        srcg, dstg = ib[gsl]
        nsrcg, ndstg = ib[1 - gsl]
        pltpu.make_async_copy(h_hbm.at[srcg.at[k]], bufs[slot],
                              gsem[slot]).wait()
        if not (first_group and k < 2):
            pltpu.make_async_copy(bufs[fslot], acc.at[dstg.at[k]],
                                  ssem[fslot]).wait()
        if k < GC - 2:
            pltpu.async_copy(h_hbm.at[srcg.at[k + 2]], bufs[fslot],
                             gsem[fslot])
        else:
            if k == GC - 2:
                pltpu.make_async_copy(src_hbm.at[w, pl.ds(0, GC)], nsrcg,
                                      isem[1 - gsl]).wait()
                pltpu.make_async_copy(dst_hbm.at[w, pl.ds(0, GC)], ndstg,
                                      isem[1 - gsl]).wait()
            pltpu.async_copy(h_hbm.at[nsrcg.at[k - (GC - 2)]], bufs[fslot],
                             gsem[fslot])
        pltpu.async_copy(bufs[slot], acc.at[dstg.at[k]], ssem[slot],
                         add=True)

    for k in range(GC):
        do_chunk(0, k, True)

    def group_pair(gp, carry):
        g1 = 1 + gp * 2
        for off in range(2):
            gsl = (1 + off) % 2
            gbase = pl.multiple_of((g1 + off) * GC, GC)
            gnext = pl.multiple_of((g1 + off + 1) * GC, GC)
            pltpu.async_copy(src_hbm.at[w, pl.ds(gnext, GC)],
                             ib[1 - gsl][0], isem[1 - gsl])
            pltpu.async_copy(dst_hbm.at[w, pl.ds(gnext, GC)],
                             ib[1 - gsl][1], isem[1 - gsl])
            del gbase
            for k in range(GC):
                do_chunk(gsl, k, False)
        return carry

    lax.fori_loop(0, (NG - 2) // 2, group_pair, 0)
    gsl = (NG - 1) % 2
    for k in range(GC):
        if k >= GC - 2:
            slot = k % 4
            fslot = (k + 2) % 4
            srcg, dstg = ib[gsl]
            pltpu.make_async_copy(h_hbm.at[srcg.at[k]], bufs[slot],
                                  gsem[slot]).wait()
            pltpu.make_async_copy(bufs[fslot], acc.at[dstg.at[k]],
                                  ssem[fslot]).wait()
            pltpu.async_copy(bufs[slot], acc.at[dstg.at[k]],
                             ssem[slot], add=True)
        else:
            do_chunk(gsl, k, False)

    for j in (NCH - 2, NCH - 1):
        sl = j % 4
        pltpu.make_async_copy(bufs[sl], acc.at[ib[0][1].at[0]],
                              ssem[sl]).wait()
    plsc.subcore_barrier()

    for k in range(RPT // CH):
        slot = k % 4
        if k >= 4:
            pltpu.make_async_copy(bufs[slot], out_hbm.at[c, pl.ds(row0, CH)],
                                  ssem[slot]).wait()
        off = pl.multiple_of(row0 + k * CH, 8)
        pltpu.sync_copy(acc.at[pl.ds(off, CH)], bufs[slot])
        pltpu.async_copy(bufs[slot], out_hbm.at[c, pl.ds(off, CH)],
                         ssem[slot])
    for k in range(RPT // CH - 4, RPT // CH):
        slot = k % 4
        pltpu.make_async_copy(bufs[slot], out_hbm.at[c, pl.ds(row0, CH)],
                              ssem[slot]).wait()


@functools.lru_cache(maxsize=None)
def _build_agg():
    mesh = plsc.VectorSubcoreMesh(core_axis_name="c", subcore_axis_name="s")
    return pl.kernel(
        _agg_body,
        out_type=jax.ShapeDtypeStruct((NC, NACC, D), jnp.float32),
        mesh=mesh,
        scratch_types=[
            pltpu.VMEM((GC, CH), jnp.int32),
            pltpu.VMEM((GC, CH), jnp.int32),
            pltpu.VMEM((GC, CH), jnp.int32),
            pltpu.VMEM((GC, CH), jnp.int32),
            pltpu.VMEM((CH, D), jnp.float32),
            pltpu.VMEM((CH, D), jnp.float32),
            pltpu.VMEM((CH, D), jnp.float32),
            pltpu.VMEM((CH, D), jnp.float32),
            pltpu.VMEM_SHARED((NACC, D), jnp.float32),
            pltpu.SemaphoreType.DMA,
            pltpu.SemaphoreType.DMA,
            pltpu.SemaphoreType.DMA,
            pltpu.SemaphoreType.DMA,
            pltpu.SemaphoreType.DMA,
            pltpu.SemaphoreType.DMA,
            pltpu.SemaphoreType.DMA,
            pltpu.SemaphoreType.DMA,
            pltpu.SemaphoreType.DMA,
            pltpu.SemaphoreType.DMA,
            pltpu.SemaphoreType.DMA,
        ],
    )


_BN_SCALE = 1.0 / (1.0 + BN_EPS) ** 0.5


def _mlp_math(x, a0, a1, onep, w1, b1, w2, b2, gam, bet):
    h = x * onep + a0 + a1
    h = jnp.maximum(jnp.dot(h, w1, preferred_element_type=jnp.float32) + b1, 0.0)
    h = jnp.maximum(jnp.dot(h, w2, preferred_element_type=jnp.float32) + b2, 0.0)
    return h * (gam * _BN_SCALE) + bet


def _mlp_block(x_ref, a0_ref, a1_ref, onep_ref, w1_ref, b1_ref, w2_ref,
               b2_ref, g_ref, be_ref, o_ref):
    o_ref[...] = _mlp_math(x_ref[...], a0_ref[...], a1_ref[...], onep_ref[...],
                           w1_ref[...], b1_ref[...], w2_ref[...], b2_ref[...],
                           g_ref[...], be_ref[...])


_ROW = lambda i: (i, 0)
_FIX = lambda i: (0, 0)


def _mlp_layer(x, a0, a1, onep, w1, b1, w2, b2, gam, bet):
    return pl.pallas_call(
        _mlp_block,
        grid=(NRB,),
        in_specs=[
            pl.BlockSpec((RB, D), _ROW),
            pl.BlockSpec((RB, D), _ROW),
            pl.BlockSpec((RB, D), _ROW),
            pl.BlockSpec((1, D), _FIX),
            pl.BlockSpec((D, D), _FIX),
            pl.BlockSpec((1, D), _FIX),
            pl.BlockSpec((D, D), _FIX),
            pl.BlockSpec((1, D), _FIX),
            pl.BlockSpec((1, D), _FIX),
            pl.BlockSpec((1, D), _FIX),
        ],
        out_specs=pl.BlockSpec((RB, D), _ROW),
        out_shape=jax.ShapeDtypeStruct((N, D), jnp.float32),
    )(x, a0, a1, onep, w1, b1, w2, b2, gam, bet)


def _final_block(x_ref, a0_ref, a1_ref, onep_ref, w1_ref, b1_ref, w2_ref,
                 b2_ref, g_ref, be_ref, batch_ref, l1w_ref, l1b_ref,
                 l2w_ref, l2b_ref, o_ref, pooled, counts):
    i = pl.program_id(0)

    @pl.when(i == 0)
    def _init():
        pooled[...] = jnp.zeros_like(pooled)
        counts[...] = jnp.zeros_like(counts)

    h = _mlp_math(x_ref[...], a0_ref[...], a1_ref[...], onep_ref[...],
                  w1_ref[...], b1_ref[...], w2_ref[...], b2_ref[...],
                  g_ref[...], be_ref[...])
    b = batch_ref[0, 0, :]
    onehot = (lax.broadcasted_iota(jnp.int32, (G, RB), 0)
              == b[None, :]).astype(jnp.float32)
    pooled[...] += jnp.dot(onehot, h, preferred_element_type=jnp.float32)
    counts[...] += jnp.sum(onehot, axis=1, keepdims=True)

    @pl.when(i == NRB - 1)
    def _head():
        p = pooled[...] / jnp.maximum(counts[...], 1.0)
        z = jnp.maximum(
            jnp.dot(p, l1w_ref[...], preferred_element_type=jnp.float32)
            + l1b_ref[...], 0.0)
        z = (jnp.dot(z, l2w_ref[...], preferred_element_type=jnp.float32)
             + l2b_ref[...])
        m = jnp.max(z, axis=-1, keepdims=True)
        e = z - m
        lse = jnp.log(jnp.sum(jnp.exp(e), axis=-1, keepdims=True))
        o_ref[...] = e - lse


def _final_layer(x, a0, a1, onep, w1, b1, w2, b2, gam, bet, batch3,
                 l1w, l1b, l2w, l2b):
    return pl.pallas_call(
        _final_block,
        grid=(NRB,),
        in_specs=[
            pl.BlockSpec((RB, D), _ROW),
            pl.BlockSpec((RB, D), _ROW),
            pl.BlockSpec((RB, D), _ROW),
            pl.BlockSpec((1, D), _FIX),
            pl.BlockSpec((D, D), _FIX),
            pl.BlockSpec((1, D), _FIX),
            pl.BlockSpec((D, D), _FIX),
            pl.BlockSpec((1, D), _FIX),
            pl.BlockSpec((1, D), _FIX),
            pl.BlockSpec((1, D), _FIX),
            pl.BlockSpec((1, 1, RB), lambda i: (i, 0, 0)),
            pl.BlockSpec((D, D), _FIX),
            pl.BlockSpec((1, D), _FIX),
            pl.BlockSpec((D, D), _FIX),
            pl.BlockSpec((1, D), _FIX),
        ],
        out_specs=pl.BlockSpec((G, D), _FIX),
        out_shape=jax.ShapeDtypeStruct((G, D), jnp.float32),
        scratch_shapes=[
            pltpu.VMEM((G, D), jnp.float32),
            pltpu.VMEM((G, D), jnp.float32),
        ],
    )(x, a0, a1, onep, w1, b1, w2, b2, gam, bet, batch3, l1w, l1b, l2w, l2b)


def kernel(x, edge_index, batch, params):
    src = edge_index[0].astype(jnp.int32).reshape(NW, EPW)
    dst = edge_index[1].astype(jnp.int32).reshape(NW, EPW)
    pad = EPWP - EPW
    src = jnp.concatenate(
        [src, jnp.zeros((NW, pad), jnp.int32)], axis=1).reshape(NW, NCH, CH)
    dst = jnp.concatenate(
        [dst, jnp.full((NW, pad), NACC - 1, jnp.int32)], axis=1
    ).reshape(NW, NCH, CH)
    batch3 = batch.astype(jnp.int32).reshape(NRB, 1, RB)
    zrows = jnp.zeros((CH, D), jnp.float32)
    agg = _build_agg()

    h = x
    n_layers = len(params['convs'])
    for li, p in enumerate(params['convs']):
        aggs = agg(h, src, dst, zrows)[:, :N, :]
        onep = jnp.broadcast_to(1.0 + p['eps'], (1, D)).astype(jnp.float32)
        w1, w2 = p['W1'], p['W2']
        b1 = p['b1'].reshape(1, D)
        b2 = p['b2'].reshape(1, D)
        gam = p['gamma'].reshape(1, D)
        bet = p['beta'].reshape(1, D)
        if li < n_layers - 1:
            h = _mlp_layer(h, aggs[0], aggs[1], onep, w1, b1, w2, b2, gam, bet)
        else:
            return _final_layer(
                h, aggs[0], aggs[1], onep, w1, b1, w2, b2, gam, bet, batch3,
                params['lin1_W'], params['lin1_b'].reshape(1, D),
                params['lin2_W'], params['lin2_b'].reshape(1, D))
